# baseline (speedup 1.0000x reference)
import jax
import jax.numpy as jnp
from jax.experimental import pallas as pl
from jax.experimental.pallas import tpu as pltpu

_H = 1024
_F = 2048
_E = 8
_T = 32
_BF = 2048
_NF = _F // _BF


def _moe_body(x_ref, gw_ref, w1_ref, w3_ref, w2_ref, out_ref, rw_ref):
    e = pl.program_id(0)
    f = pl.program_id(1)

    @pl.when((e == 0) & (f == 0))
    def _routing():
        x = x_ref[...]
        logits = jax.lax.dot_general(
            x, gw_ref[...], (((1,), (1,)), ((), ())),
            preferred_element_type=jnp.float32)
        col = jax.lax.broadcasted_iota(jnp.int32, logits.shape, 1)
        v0 = jnp.max(logits, axis=-1, keepdims=True)
        i0 = jnp.argmax(logits, axis=-1)[:, None]
        hot0 = col == i0
        masked = jnp.where(hot0, -jnp.inf, logits)
        v1 = jnp.max(masked, axis=-1, keepdims=True)
        i1 = jnp.argmax(masked, axis=-1)[:, None]
        hot1 = col == i1
        r = jnp.exp(v1 - v0)
        w_hi = 1.0 / (1.0 + r)
        w_lo = r / (1.0 + r)
        rw_ref[...] = jnp.where(hot0, w_hi, 0.0) + jnp.where(hot1, w_lo, 0.0)

    x = x_ref[...]
    h1 = jax.lax.dot_general(x, w1_ref[0, 0], (((1,), (1,)), ((), ())),
                             preferred_element_type=jnp.float32)
    h3 = jax.lax.dot_general(x, w3_ref[0, 0], (((1,), (1,)), ((), ())),
                             preferred_element_type=jnp.float32)
    act = h1 * jax.nn.sigmoid(h1) * h3
    contrib = jax.lax.dot_general(act, w2_ref[0], (((1,), (1,)), ((), ())),
                                  preferred_element_type=jnp.float32)
    onehot = (jax.lax.broadcasted_iota(jnp.int32, (_E, 1), 0) == e
              ).astype(jnp.float32)
    scale = jax.lax.dot_general(rw_ref[...], onehot, (((1,), (0,)), ((), ())),
                                preferred_element_type=jnp.float32)
    contrib = contrib * scale

    @pl.when((e == 0) & (f == 0))
    def _init():
        out_ref[...] = contrib

    @pl.when(~((e == 0) & (f == 0)))
    def _acc():
        out_ref[...] += contrib


@jax.jit
def kernel(x, gate_w, w13, w2):
    w13r = w13.reshape(_E, 2, _F, _H)
    grid = (_E, _NF)
    return pl.pallas_call(
        _moe_body,
        grid=grid,
        in_specs=[
            pl.BlockSpec((_T, _H), lambda e, f: (0, 0)),
            pl.BlockSpec((_E, _H), lambda e, f: (0, 0)),
            pl.BlockSpec((1, 1, _BF, _H), lambda e, f: (e, 0, f, 0)),
            pl.BlockSpec((1, 1, _BF, _H), lambda e, f: (e, 1, f, 0)),
            pl.BlockSpec((1, _H, _BF), lambda e, f: (e, 0, f)),
        ],
        out_specs=pl.BlockSpec((_T, _H), lambda e, f: (0, 0)),
        out_shape=jax.ShapeDtypeStruct((_T, _H), jnp.float32),
        scratch_shapes=[pltpu.VMEM((_T, _E), jnp.float32)],
        compiler_params=pltpu.CompilerParams(
            dimension_semantics=("arbitrary", "arbitrary"),
        ),
    )(x, gate_w, w13r, w13r, w2)


# speedup vs baseline: 1.0263x; 1.0263x over previous
import jax
import jax.numpy as jnp
from jax.experimental import pallas as pl
from jax.experimental.pallas import tpu as pltpu

_H = 1024
_F = 2048
_E = 8
_T = 32
_BF = 1024
_NF = _F // _BF


def _moe_body(x_ref, gw_ref, w1_ref, w3_ref, w2_ref, out_ref, act_ref, rw_ref):
    e = pl.program_id(0)
    f = pl.program_id(1)

    @pl.when((e == 0) & (f == 0))
    def _routing():
        x = x_ref[...]
        logits = jax.lax.dot_general(
            x, gw_ref[...], (((1,), (1,)), ((), ())),
            preferred_element_type=jnp.float32)
        col = jax.lax.broadcasted_iota(jnp.int32, logits.shape, 1)
        v0 = jnp.max(logits, axis=-1, keepdims=True)
        i0 = jnp.argmax(logits, axis=-1)[:, None]
        hot0 = col == i0
        masked = jnp.where(hot0, -jnp.inf, logits)
        v1 = jnp.max(masked, axis=-1, keepdims=True)
        i1 = jnp.argmax(masked, axis=-1)[:, None]
        hot1 = col == i1
        r = jnp.exp(v1 - v0)
        w_hi = 1.0 / (1.0 + r)
        w_lo = r / (1.0 + r)
        rw_ref[...] = jnp.where(hot0, w_hi, 0.0) + jnp.where(hot1, w_lo, 0.0)

    x = x_ref[...]
    h1 = jax.lax.dot_general(x, w1_ref[0, 0], (((1,), (1,)), ((), ())),
                             preferred_element_type=jnp.float32)
    h3 = jax.lax.dot_general(x, w3_ref[0, 0], (((1,), (1,)), ((), ())),
                             preferred_element_type=jnp.float32)
    act_ref[:, pl.ds(f * _BF, _BF)] = h1 * jax.nn.sigmoid(h1) * h3

    @pl.when(f == _NF - 1)
    def _combine():
        act = act_ref[...]
        contrib = jax.lax.dot_general(
            act, w2_ref[0], (((1,), (1,)), ((), ())),
            preferred_element_type=jnp.float32)
        onehot = (jax.lax.broadcasted_iota(jnp.int32, (_E, 1), 0) == e
                  ).astype(jnp.float32)
        scale = jax.lax.dot_general(
            rw_ref[...], onehot, (((1,), (0,)), ((), ())),
            preferred_element_type=jnp.float32)
        contrib = contrib * scale

        @pl.when(e == 0)
        def _init():
            out_ref[...] = contrib

        @pl.when(e != 0)
        def _acc():
            out_ref[...] += contrib


@jax.jit
def kernel(x, gate_w, w13, w2):
    w13r = w13.reshape(_E, 2, _F, _H)
    grid = (_E, _NF)
    return pl.pallas_call(
        _moe_body,
        grid=grid,
        in_specs=[
            pl.BlockSpec((_T, _H), lambda e, f: (0, 0)),
            pl.BlockSpec((_E, _H), lambda e, f: (0, 0)),
            pl.BlockSpec((1, 1, _BF, _H), lambda e, f: (e, 0, f, 0)),
            pl.BlockSpec((1, 1, _BF, _H), lambda e, f: (e, 1, f, 0)),
            pl.BlockSpec((1, _H, _F), lambda e, f: (e, 0, 0)),
        ],
        out_specs=pl.BlockSpec((_T, _H), lambda e, f: (0, 0)),
        out_shape=jax.ShapeDtypeStruct((_T, _H), jnp.float32),
        scratch_shapes=[
            pltpu.VMEM((_T, _F), jnp.float32),
            pltpu.VMEM((_T, _E), jnp.float32),
        ],
        compiler_params=pltpu.CompilerParams(
            dimension_semantics=("arbitrary", "arbitrary"),
        ),
    )(x, gate_w, w13r, w13r, w2)


# speedup vs baseline: 1.0425x; 1.0158x over previous
import jax
import jax.numpy as jnp
from jax.experimental import pallas as pl
from jax.experimental.pallas import tpu as pltpu

_H = 1024
_F = 2048
_E = 8
_T = 32
_BF = 1024
_NF = _F // _BF
_HB = _BF // 2
_HH = _H // 2


def _moe_body(x_ref, gw_ref, w1a_ref, w1b_ref, w3a_ref, w3b_ref,
              w2a_ref, w2b_ref, out_ref, rw_ref):
    e = pl.program_id(0)
    f = pl.program_id(1)

    @pl.when((e == 0) & (f == 0))
    def _routing():
        x = x_ref[...]
        logits = jax.lax.dot_general(
            x, gw_ref[...], (((1,), (1,)), ((), ())),
            preferred_element_type=jnp.float32)
        col = jax.lax.broadcasted_iota(jnp.int32, logits.shape, 1)
        v0 = jnp.max(logits, axis=-1, keepdims=True)
        i0 = jnp.argmax(logits, axis=-1)[:, None]
        hot0 = col == i0
        masked = jnp.where(hot0, -jnp.inf, logits)
        v1 = jnp.max(masked, axis=-1, keepdims=True)
        i1 = jnp.argmax(masked, axis=-1)[:, None]
        hot1 = col == i1
        r = jnp.exp(v1 - v0)
        w_hi = 1.0 / (1.0 + r)
        w_lo = r / (1.0 + r)
        rw_ref[...] = jnp.where(hot0, w_hi, 0.0) + jnp.where(hot1, w_lo, 0.0)

    x = x_ref[...]

    def mm(a, b):
        return jax.lax.dot_general(a, b, (((1,), (1,)), ((), ())),
                                   preferred_element_type=jnp.float32)

    h1a = mm(x, w1a_ref[0, 0, 0])
    h1b = mm(x, w1b_ref[0, 0, 0])
    h3a = mm(x, w3a_ref[0, 0, 0])
    h3b = mm(x, w3b_ref[0, 0, 0])
    act = jnp.concatenate(
        [h1a * jax.nn.sigmoid(h1a) * h3a,
         h1b * jax.nn.sigmoid(h1b) * h3b], axis=1)
    ca = mm(act, w2a_ref[0, 0])
    cb = mm(act, w2b_ref[0, 0])
    onehot = (jax.lax.broadcasted_iota(jnp.int32, (_E, 1), 0) == e
              ).astype(jnp.float32)
    scale = jax.lax.dot_general(rw_ref[...], onehot, (((1,), (0,)), ((), ())),
                                preferred_element_type=jnp.float32)
    contrib = jnp.concatenate([ca, cb], axis=1) * scale

    @pl.when((e == 0) & (f == 0))
    def _init():
        out_ref[...] = contrib

    @pl.when(~((e == 0) & (f == 0)))
    def _acc():
        out_ref[...] += contrib


@jax.jit
def kernel(x, gate_w, w13, w2):
    w13r = w13.reshape(_E, 2, _F // _HB, _HB, _H)
    w2r = w2.reshape(_E, 2, _HH, _F)
    grid = (_E, _NF)
    return pl.pallas_call(
        _moe_body,
        grid=grid,
        in_specs=[
            pl.BlockSpec((_T, _H), lambda e, f: (0, 0)),
            pl.BlockSpec((_E, _H), lambda e, f: (0, 0)),
            pl.BlockSpec((1, 1, 1, _HB, _H), lambda e, f: (e, 0, 2 * f, 0, 0)),
            pl.BlockSpec((1, 1, 1, _HB, _H), lambda e, f: (e, 0, 2 * f + 1, 0, 0)),
            pl.BlockSpec((1, 1, 1, _HB, _H), lambda e, f: (e, 1, 2 * f, 0, 0)),
            pl.BlockSpec((1, 1, 1, _HB, _H), lambda e, f: (e, 1, 2 * f + 1, 0, 0)),
            pl.BlockSpec((1, 1, _HH, _BF), lambda e, f: (e, 0, 0, f)),
            pl.BlockSpec((1, 1, _HH, _BF), lambda e, f: (e, 1, 0, f)),
        ],
        out_specs=pl.BlockSpec((_T, _H), lambda e, f: (0, 0)),
        out_shape=jax.ShapeDtypeStruct((_T, _H), jnp.float32),
        scratch_shapes=[pltpu.VMEM((_T, _E), jnp.float32)],
        compiler_params=pltpu.CompilerParams(
            dimension_semantics=("arbitrary", "arbitrary"),
        ),
    )(x, gate_w, w13r, w13r, w13r, w13r, w2r, w2r)


# speedup vs baseline: 1.0861x; 1.0418x over previous
import jax
import jax.numpy as jnp
from jax.experimental import pallas as pl
from jax.experimental.pallas import tpu as pltpu

_H = 1024
_F = 2048
_E = 8
_T = 32
_BF = 1024
_NF = _F // _BF


def _moe_body(x_ref, gw_ref, w13_ref, w2_ref, out_ref, rw_ref):
    e = pl.program_id(0)
    f = pl.program_id(1)

    @pl.when((e == 0) & (f == 0))
    def _routing():
        x = x_ref[...]
        logits = jax.lax.dot_general(
            x, gw_ref[...], (((1,), (1,)), ((), ())),
            preferred_element_type=jnp.float32)
        col = jax.lax.broadcasted_iota(jnp.int32, logits.shape, 1)
        v0 = jnp.max(logits, axis=-1, keepdims=True)
        i0 = jnp.argmax(logits, axis=-1)[:, None]
        hot0 = col == i0
        masked = jnp.where(hot0, -jnp.inf, logits)
        v1 = jnp.max(masked, axis=-1, keepdims=True)
        i1 = jnp.argmax(masked, axis=-1)[:, None]
        hot1 = col == i1
        r = jnp.exp(v1 - v0)
        w_hi = 1.0 / (1.0 + r)
        w_lo = r / (1.0 + r)
        rw_ref[...] = jnp.where(hot0, w_hi, 0.0) + jnp.where(hot1, w_lo, 0.0)

    x = x_ref[...]

    def mm(a, b):
        return jax.lax.dot_general(a, b, (((1,), (1,)), ((), ())),
                                   preferred_element_type=jnp.float32)

    h1 = mm(x, w13_ref[0, 0])
    h3 = mm(x, w13_ref[0, 1])
    act = h1 * jax.nn.sigmoid(h1) * h3
    contrib = mm(act, w2_ref[0])
    onehot = (jax.lax.broadcasted_iota(jnp.int32, (_E, 1), 0) == e
              ).astype(jnp.float32)
    scale = jax.lax.dot_general(rw_ref[...], onehot, (((1,), (0,)), ((), ())),
                                preferred_element_type=jnp.float32)
    contrib = contrib * scale

    @pl.when((e == 0) & (f == 0))
    def _init():
        out_ref[...] = contrib

    @pl.when(~((e == 0) & (f == 0)))
    def _acc():
        out_ref[...] += contrib


@jax.jit
def kernel(x, gate_w, w13, w2):
    w13r = w13.reshape(_E, 2, _F, _H)
    grid = (_E, _NF)
    return pl.pallas_call(
        _moe_body,
        grid=grid,
        in_specs=[
            pl.BlockSpec((_T, _H), lambda e, f: (0, 0)),
            pl.BlockSpec((_E, _H), lambda e, f: (0, 0)),
            pl.BlockSpec((1, 2, _BF, _H), lambda e, f: (e, 0, f, 0)),
            pl.BlockSpec((1, _H, _BF), lambda e, f: (e, 0, f)),
        ],
        out_specs=pl.BlockSpec((_T, _H), lambda e, f: (0, 0)),
        out_shape=jax.ShapeDtypeStruct((_T, _H), jnp.float32),
        scratch_shapes=[pltpu.VMEM((_T, _E), jnp.float32)],
        compiler_params=pltpu.CompilerParams(
            dimension_semantics=("arbitrary", "arbitrary"),
        ),
    )(x, gate_w, w13r, w2)
